# Initial kernel scaffold; baseline (speedup 1.0000x reference)
import jax
import jax.numpy as jnp
from jax.experimental import pallas as pl


def kernel(current_state, neighbor_activity, expert_out_0, expert_out_1, expert_out_2, ln_gamma, ln_beta, W1, b1, W2, b2):
    raise NotImplementedError("write your pallas kernel here")


# speedup vs baseline: 1.5406x; 1.5406x over previous
import jax
import jax.numpy as jnp
from jax.experimental import pallas as pl

B = 8192
D = 1024
H = 256
E = 3
EP = 128
BM = 512


def _fused_kernel(cs_ref, na_ref, e0_ref, e1_ref, e2_ref, gamma_ref, beta_ref,
                  w1a_ref, w1b_ref, b1_ref, w2p_ref, b2p_ref,
                  out_ref, wts_ref):
    cs = cs_ref[...]
    mu = jnp.mean(cs, axis=1, keepdims=True)
    xc = cs - mu
    var = jnp.mean(xc * xc, axis=1, keepdims=True)
    ns = xc * jax.lax.rsqrt(var + 1e-5) * gamma_ref[...] + beta_ref[...]
    h = (jnp.dot(ns, w1a_ref[...], preferred_element_type=jnp.float32)
         + jnp.dot(na_ref[...], w1b_ref[...], preferred_element_type=jnp.float32)
         + b1_ref[...])
    h = 0.5 * h * (1.0 + jax.lax.erf(h * 0.7071067811865476))
    logits = jnp.dot(h, w2p_ref[...], preferred_element_type=jnp.float32) + b2p_ref[...]
    m = jnp.max(logits, axis=1, keepdims=True)
    ex = jnp.exp(logits - m)
    w = ex / jnp.sum(ex, axis=1, keepdims=True)
    wts_ref[...] = w
    out_ref[...] = (w[:, 0:1] * e0_ref[...]
                    + w[:, 1:2] * e1_ref[...]
                    + w[:, 2:3] * e2_ref[...])


def kernel(current_state, neighbor_activity, expert_out_0, expert_out_1, expert_out_2, ln_gamma, ln_beta, W1, b1, W2, b2):
    gamma = ln_gamma.reshape(1, D)
    beta = ln_beta.reshape(1, D)
    w1a = W1[:D]
    w1b = W1[D:]
    b1r = b1.reshape(1, H)
    w2p = jnp.zeros((H, EP), jnp.float32).at[:, :E].set(W2)
    b2p = jnp.full((1, EP), -1e30, jnp.float32).at[0, :E].set(b2)

    grid = (B // BM,)
    row = lambda i: (i, 0)
    rep = lambda i: (0, 0)
    out, wts = pl.pallas_call(
        _fused_kernel,
        grid=grid,
        in_specs=[
            pl.BlockSpec((BM, D), row),
            pl.BlockSpec((BM, D), row),
            pl.BlockSpec((BM, D), row),
            pl.BlockSpec((BM, D), row),
            pl.BlockSpec((BM, D), row),
            pl.BlockSpec((1, D), rep),
            pl.BlockSpec((1, D), rep),
            pl.BlockSpec((D, H), rep),
            pl.BlockSpec((D, H), rep),
            pl.BlockSpec((1, H), rep),
            pl.BlockSpec((H, EP), rep),
            pl.BlockSpec((1, EP), rep),
        ],
        out_specs=[
            pl.BlockSpec((BM, D), row),
            pl.BlockSpec((BM, EP), row),
        ],
        out_shape=[
            jax.ShapeDtypeStruct((B, D), jnp.float32),
            jax.ShapeDtypeStruct((B, EP), jnp.float32),
        ],
    )(current_state, neighbor_activity, expert_out_0, expert_out_1,
      expert_out_2, gamma, beta, w1a, w1b, b1r, w2p, b2p)
    return out, wts[:, :E]


# speedup vs baseline: 1.5543x; 1.0088x over previous
import jax
import jax.numpy as jnp
from jax.experimental import pallas as pl

B = 8192
D = 1024
H = 256
E = 3
EP = 128
BM = 512


def _fused_kernel(cs_ref, na_ref, e0_ref, e1_ref, e2_ref, gamma_ref, beta_ref,
                  w1a_ref, w1b_ref, b1_ref, w2p_ref, b2p_ref,
                  out_ref, wts_ref):
    cs = cs_ref[...]
    mu = jnp.mean(cs, axis=1, keepdims=True)
    xc = cs - mu
    var = jnp.mean(xc * xc, axis=1, keepdims=True)
    ns = xc * jax.lax.rsqrt(var + 1e-5) * gamma_ref[...] + beta_ref[...]
    h = (jnp.dot(ns.astype(jnp.bfloat16), w1a_ref[...],
                 preferred_element_type=jnp.float32)
         + jnp.dot(na_ref[...].astype(jnp.bfloat16), w1b_ref[...],
                   preferred_element_type=jnp.float32)
         + b1_ref[...])
    h = 0.5 * h * (1.0 + jax.lax.erf(h * 0.7071067811865476))
    logits = jnp.dot(h, w2p_ref[...], preferred_element_type=jnp.float32) + b2p_ref[...]
    m = jnp.max(logits, axis=1, keepdims=True)
    ex = jnp.exp(logits - m)
    w = ex / jnp.sum(ex, axis=1, keepdims=True)
    wts_ref[...] = w
    out_ref[...] = (w[:, 0:1] * e0_ref[...]
                    + w[:, 1:2] * e1_ref[...]
                    + w[:, 2:3] * e2_ref[...])


def kernel(current_state, neighbor_activity, expert_out_0, expert_out_1, expert_out_2, ln_gamma, ln_beta, W1, b1, W2, b2):
    gamma = ln_gamma.reshape(1, D)
    beta = ln_beta.reshape(1, D)
    w1a = W1[:D].astype(jnp.bfloat16)
    w1b = W1[D:].astype(jnp.bfloat16)
    b1r = b1.reshape(1, H)
    w2p = jnp.zeros((H, EP), jnp.float32).at[:, :E].set(W2)
    b2p = jnp.full((1, EP), -1e30, jnp.float32).at[0, :E].set(b2)

    grid = (B // BM,)
    row = lambda i: (i, 0)
    rep = lambda i: (0, 0)
    out, wts = pl.pallas_call(
        _fused_kernel,
        grid=grid,
        in_specs=[
            pl.BlockSpec((BM, D), row),
            pl.BlockSpec((BM, D), row),
            pl.BlockSpec((BM, D), row),
            pl.BlockSpec((BM, D), row),
            pl.BlockSpec((BM, D), row),
            pl.BlockSpec((1, D), rep),
            pl.BlockSpec((1, D), rep),
            pl.BlockSpec((D, H), rep),
            pl.BlockSpec((D, H), rep),
            pl.BlockSpec((1, H), rep),
            pl.BlockSpec((H, EP), rep),
            pl.BlockSpec((1, EP), rep),
        ],
        out_specs=[
            pl.BlockSpec((BM, D), row),
            pl.BlockSpec((BM, EP), row),
        ],
        out_shape=[
            jax.ShapeDtypeStruct((B, D), jnp.float32),
            jax.ShapeDtypeStruct((B, EP), jnp.float32),
        ],
    )(current_state, neighbor_activity, expert_out_0, expert_out_1,
      expert_out_2, gamma, beta, w1a, w1b, b1r, w2p, b2p)
    return out, wts[:, :E]
